# Initial kernel scaffold; baseline (speedup 1.0000x reference)
import jax
import jax.numpy as jnp
from jax.experimental import pallas as pl


def kernel(pred_logits, pred_boxes, target_sizes):
    raise NotImplementedError("write your pallas kernel here")


# speedup vs baseline: 16.9305x; 16.9305x over previous
import functools

import jax
import jax.numpy as jnp
from jax.experimental import pallas as pl

B, N, C = 16, 20000, 91
K = 100
ROW_BLK = 2048


def _rowmax_kernel(x_ref, o_ref):
    o_ref[...] = jnp.max(x_ref[...], axis=-1)[:, None, :]


@jax.jit
def _rowmax(pred_logits):
    nb = pl.cdiv(N, ROW_BLK)
    out = pl.pallas_call(
        _rowmax_kernel,
        grid=(B, nb),
        in_specs=[pl.BlockSpec((1, ROW_BLK, C), lambda b, i: (b, i, 0))],
        out_specs=pl.BlockSpec((1, 1, ROW_BLK), lambda b, i: (b * nb + i, 0, 0)),
        out_shape=jax.ShapeDtypeStruct((B * nb, 1, ROW_BLK), jnp.float32),
    )(pred_logits)
    return out.reshape(B, nb * ROW_BLK)[:, :N]


@jax.jit
def kernel(pred_logits, pred_boxes, target_sizes):
    rowmax = _rowmax(pred_logits)
    _, rids = jax.lax.top_k(rowmax, K)
    cand = jnp.take_along_axis(pred_logits, rids[:, :, None], axis=1)
    vals, idx = jax.lax.top_k(cand.reshape(B, K * C), K)
    local_row = idx // C
    labels = idx % C
    topk_boxes = jnp.take_along_axis(rids, local_row, axis=1)
    scores = jax.nn.sigmoid(vals)

    gather_idx = jnp.broadcast_to(topk_boxes[:, :, None], (B, K, 4))
    bx = jnp.take_along_axis(pred_boxes, gather_idx, axis=1)
    cx, cy, w, h = jnp.split(bx, 4, axis=-1)
    boxes = jnp.concatenate(
        [cx - 0.5 * w, cy - 0.5 * h, cx + 0.5 * w, cy + 0.5 * h], axis=-1
    )
    img_h = target_sizes[:, 0]
    img_w = target_sizes[:, 1]
    scale_fct = jnp.stack([img_w, img_h, img_w, img_h], axis=1)
    boxes = boxes * scale_fct[:, None, :]
    return scores, labels, boxes
